# baseline (speedup 1.0000x reference)
import functools
import math

import jax
import jax.numpy as jnp
from jax import lax
from jax.experimental import pallas as pl
from jax.experimental.pallas import tpu as pltpu
from jax.experimental.pallas import tpu_sc as plsc

_D = 64
_NC, _NS = 2, 16
_NW = _NC * _NS
_LANES = 16
_SCALE = math.sqrt(_D)
_TI = 128


@jax.jit
def _embed_lookup(tok_t, table_pairs):
    hist, nrows = tok_t.shape
    i_per_w = nrows // _NW
    ib_per_w = i_per_w // _TI
    nblocks = hist * ib_per_w
    mesh = plsc.VectorSubcoreMesh(
        core_axis_name="c", subcore_axis_name="s",
        num_cores=_NC, num_subcores=_NS)

    @functools.partial(
        pl.kernel,
        out_type=jax.ShapeDtypeStruct((hist, _D, nrows), jnp.float32),
        mesh=mesh,
        compiler_params=pltpu.CompilerParams(
            use_tc_tiling_on_sc=True, needs_layout_passes=False),
        scratch_types=[
            pltpu.VMEM((2, _TI), jnp.int32),
            pltpu.VMEM((2, _TI), jnp.int32),
            pltpu.VMEM((2, _TI, 2 * _D), jnp.float32),
            pltpu.VMEM((2, _D, _TI), jnp.float32),
            pltpu.SemaphoreType.DMA,
            pltpu.SemaphoreType.DMA,
            pltpu.SemaphoreType.DMA,
            pltpu.SemaphoreType.DMA,
        ],
    )
    def k(tok_hbm, tab_hbm, out_hbm, pidx_v, hoff_v, rows_v, trans_v, *sems):
        gsems = sems[:2]
        osems = sems[2:]
        wid = lax.axis_index("s") * _NC + lax.axis_index("c")
        i_base = wid * i_per_w

        def start_gather(buf, blk, gsem):
            j = lax.shift_right_logical(blk, 2)
            ib = lax.bitwise_and(blk, ib_per_w - 1)
            i0 = i_base + ib * _TI
            pltpu.sync_copy(tok_hbm.at[j, pl.ds(i0, _TI)], pidx_v.at[buf])
            for l in range(_TI // _LANES):
                sl = pl.ds(l * _LANES, _LANES)
                t = pidx_v[buf, sl]
                hoff_v[buf, sl] = lax.shift_left(
                    lax.bitwise_and(t, 1), _D.bit_length() - 1)
                pidx_v[buf, sl] = lax.shift_right_logical(t, 1)
            pltpu.async_copy(tab_hbm.at[pidx_v.at[buf]], rows_v.at[buf], gsem)

        start_gather(0, 0, gsems[0])

        @pl.loop(0, nblocks, step=2)
        def _outer(G):
            for b in range(2):
                g = G + b
                nb = 1 - b

                @pl.when(g + 1 < nblocks)
                def _start_next():
                    @pl.when(g >= 1)
                    def _drain():
                        pltpu.make_async_copy(
                            trans_v.at[nb],
                            out_hbm.at[0, :, pl.ds(i_base, _TI)],
                            osems[nb]).wait()
                    start_gather(nb, g + 1, gsems[nb])

                pltpu.make_async_copy(
                    tab_hbm.at[pidx_v.at[b]], rows_v.at[b], gsems[b]).wait()

                lanes = lax.iota(jnp.int32, _LANES)

                @pl.loop(0, _D, unroll=4)
                def _feat(f):
                    for l in range(_TI // _LANES):
                        rid = l * _LANES + lanes
                        cid = hoff_v[b, pl.ds(l * _LANES, _LANES)] + f
                        val = plsc.load_gather(rows_v.at[b], [rid, cid])
                        trans_v[b, f, pl.ds(l * _LANES, _LANES)] = val * _SCALE

                j = lax.shift_right_logical(g, 2)
                ib = lax.bitwise_and(g, ib_per_w - 1)
                i0 = i_base + ib * _TI
                pltpu.async_copy(
                    trans_v.at[b], out_hbm.at[j, :, pl.ds(i0, _TI)],
                    osems[b])

        for b in range(2):
            pltpu.make_async_copy(
                trans_v.at[b], out_hbm.at[0, :, pl.ds(i_base, _TI)],
                osems[b]).wait()

    return k(tok_t, table_pairs)


def kernel(tokens, table):
    tok_t = tokens.astype(jnp.int32).T
    table_pairs = table.reshape(table.shape[0] // 2, 2 * _D)
    out = _embed_lookup(tok_t, table_pairs)
    return jnp.transpose(out, (2, 0, 1))


# speedup vs baseline: 1.6560x; 1.6560x over previous
import functools
import math

import jax
import jax.numpy as jnp
from jax import lax
from jax.experimental import pallas as pl
from jax.experimental.pallas import tpu as pltpu
from jax.experimental.pallas import tpu_sc as plsc

_D = 64
_NC, _NS = 2, 16
_NW = _NC * _NS
_LANES = 16
_SCALE = math.sqrt(_D)
_TI = 128


@jax.jit
def _embed_lookup(tok_t, table_pad):
    hist, nrows = tok_t.shape
    i_per_w = nrows // _NW
    ib_per_w = i_per_w // _TI
    nblocks = hist * ib_per_w
    mesh = plsc.VectorSubcoreMesh(
        core_axis_name="c", subcore_axis_name="s",
        num_cores=_NC, num_subcores=_NS)

    @functools.partial(
        pl.kernel,
        out_type=jax.ShapeDtypeStruct((hist, _D, nrows), jnp.float32),
        mesh=mesh,
        compiler_params=pltpu.CompilerParams(
            use_tc_tiling_on_sc=True, needs_layout_passes=False),
        scratch_types=[
            pltpu.VMEM((2, _TI), jnp.int32),
            pltpu.VMEM((2, _TI, 2 * _D), jnp.float32),
            pltpu.VMEM((2, _D, _TI), jnp.float32),
            pltpu.SemaphoreType.DMA,
            pltpu.SemaphoreType.DMA,
            pltpu.SemaphoreType.DMA,
            pltpu.SemaphoreType.DMA,
        ],
    )
    def k(tok_hbm, tab_hbm, out_hbm, idx_v, rows_v, trans_v, *sems):
        gsems = sems[:2]
        osems = sems[2:]
        wid = lax.axis_index("s") * _NC + lax.axis_index("c")
        i_base = wid * i_per_w

        def start_gather(buf, blk, gsem):
            j = lax.shift_right_logical(blk, 2)
            ib = lax.bitwise_and(blk, ib_per_w - 1)
            i0 = i_base + ib * _TI
            pltpu.sync_copy(tok_hbm.at[j, pl.ds(i0, _TI)], idx_v.at[buf])
            pltpu.async_copy(tab_hbm.at[idx_v.at[buf]], rows_v.at[buf], gsem)

        start_gather(0, 0, gsems[0])

        lanes = lax.iota(jnp.int32, _LANES)

        @pl.loop(0, nblocks, step=2)
        def _outer(G):
            for b in range(2):
                g = G + b
                nb = 1 - b

                @pl.when(g + 1 < nblocks)
                def _start_next():
                    @pl.when(g >= 1)
                    def _drain():
                        pltpu.make_async_copy(
                            trans_v.at[nb],
                            out_hbm.at[0, :, pl.ds(i_base, _TI)],
                            osems[nb]).wait()
                    start_gather(nb, g + 1, gsems[nb])

                pltpu.make_async_copy(
                    tab_hbm.at[idx_v.at[b]], rows_v.at[b], gsems[b]).wait()

                @pl.loop(0, _TI, unroll=4)
                def _tok(r):
                    cid = jnp.full((_LANES,), 0, jnp.int32) + r
                    for l in range(_D // _LANES):
                        rid = l * _LANES + lanes
                        val = rows_v[b, r, pl.ds(l * _LANES, _LANES)] * _SCALE
                        plsc.store_scatter(trans_v.at[b], [rid, cid], val)

                j = lax.shift_right_logical(g, 2)
                ib = lax.bitwise_and(g, ib_per_w - 1)
                i0 = i_base + ib * _TI
                pltpu.async_copy(
                    trans_v.at[b], out_hbm.at[j, :, pl.ds(i0, _TI)],
                    osems[b])

        for b in range(2):
            pltpu.make_async_copy(
                trans_v.at[b], out_hbm.at[0, :, pl.ds(i_base, _TI)],
                osems[b]).wait()

    return k(tok_t, table_pad)


def kernel(tokens, table):
    tok_t = tokens.astype(jnp.int32).T
    table_pad = jnp.pad(table, ((0, 0), (0, _D)))
    out = _embed_lookup(tok_t, table_pad)
    return jnp.transpose(out, (2, 0, 1))


# speedup vs baseline: 2.3260x; 1.4046x over previous
import functools
import math

import jax
import jax.numpy as jnp
from jax import lax
from jax.experimental import pallas as pl
from jax.experimental.pallas import tpu as pltpu
from jax.experimental.pallas import tpu_sc as plsc

_D = 64
_NC, _NS = 2, 16
_NW = _NC * _NS
_LANES = 16
_SCALE = math.sqrt(_D)
_TI = 128


@jax.jit
def _embed_lookup(tok_t, table_pad):
    hist, nrows = tok_t.shape
    i_per_w = nrows // _NW
    ib_per_w = i_per_w // _TI
    nblocks = hist * ib_per_w
    mesh = plsc.VectorSubcoreMesh(
        core_axis_name="c", subcore_axis_name="s",
        num_cores=_NC, num_subcores=_NS)

    @functools.partial(
        pl.kernel,
        out_type=jax.ShapeDtypeStruct((hist, _D, nrows), jnp.float32),
        mesh=mesh,
        compiler_params=pltpu.CompilerParams(
            use_tc_tiling_on_sc=True, needs_layout_passes=False),
        scratch_types=[
            pltpu.VMEM((2, _TI), jnp.int32),
            pltpu.VMEM((2, _TI, 2 * _D), jnp.float32),
            pltpu.VMEM((2, _D, _TI), jnp.float32),
            pltpu.SemaphoreType.DMA,
            pltpu.SemaphoreType.DMA,
            pltpu.SemaphoreType.DMA,
            pltpu.SemaphoreType.DMA,
        ],
    )
    def k(tok_hbm, tab_hbm, out_hbm, idx_v, rows_v, trans_v, *sems):
        gsems = sems[:2]
        osems = sems[2:]
        wid = lax.axis_index("s") * _NC + lax.axis_index("c")
        i_base = wid * i_per_w

        def start_gather(buf, blk, gsem):
            j = lax.shift_right_logical(blk, 2)
            ib = lax.bitwise_and(blk, ib_per_w - 1)
            i0 = i_base + ib * _TI
            pltpu.sync_copy(tok_hbm.at[j, pl.ds(i0, _TI)], idx_v.at[buf])
            pltpu.async_copy(tab_hbm.at[idx_v.at[buf]], rows_v.at[buf], gsem)

        start_gather(0, 0, gsems[0])

        lanes = lax.iota(jnp.int32, _LANES)

        @pl.loop(0, nblocks, step=2)
        def _outer(G):
            for b in range(2):
                g = G + b
                nb = 1 - b

                @pl.when(g + 1 < nblocks)
                def _start_next():
                    @pl.when(g >= 1)
                    def _drain():
                        pltpu.make_async_copy(
                            trans_v.at[nb],
                            out_hbm.at[0, :, pl.ds(i_base, _TI)],
                            osems[nb]).wait()
                    start_gather(nb, g + 1, gsems[nb])

                pltpu.make_async_copy(
                    tab_hbm.at[idx_v.at[b]], rows_v.at[b], gsems[b]).wait()

                @plsc.parallel_loop(0, _TI, unroll=8)
                def _tok(r):
                    cid = jnp.full((_LANES,), 0, jnp.int32) + r
                    for l in range(_D // _LANES):
                        rid = l * _LANES + lanes
                        val = rows_v[b, r, pl.ds(l * _LANES, _LANES)] * _SCALE
                        plsc.store_scatter(trans_v.at[b], [rid, cid], val)

                j = lax.shift_right_logical(g, 2)
                ib = lax.bitwise_and(g, ib_per_w - 1)
                i0 = i_base + ib * _TI
                pltpu.async_copy(
                    trans_v.at[b], out_hbm.at[j, :, pl.ds(i0, _TI)],
                    osems[b])

        for b in range(2):
            pltpu.make_async_copy(
                trans_v.at[b], out_hbm.at[0, :, pl.ds(i_base, _TI)],
                osems[b]).wait()

    return k(tok_t, table_pad)


def kernel(tokens, table):
    tok_t = tokens.astype(jnp.int32).T
    table_pad = jnp.pad(table, ((0, 0), (0, _D)))
    out = _embed_lookup(tok_t, table_pad)
    return jnp.transpose(out, (2, 0, 1))


# speedup vs baseline: 2.3400x; 1.0060x over previous
import functools
import math

import jax
import jax.numpy as jnp
from jax import lax
from jax.experimental import pallas as pl
from jax.experimental.pallas import tpu as pltpu
from jax.experimental.pallas import tpu_sc as plsc

_D = 64
_NC, _NS = 2, 16
_NW = _NC * _NS
_LANES = 16
_SCALE = math.sqrt(_D)
_TI = 128


@jax.jit
def _embed_lookup(tok_t, table_pad):
    hist, nrows = tok_t.shape
    i_per_w = nrows // _NW
    ib_per_w = i_per_w // _TI
    nblocks = hist * ib_per_w
    mesh = plsc.VectorSubcoreMesh(
        core_axis_name="c", subcore_axis_name="s",
        num_cores=_NC, num_subcores=_NS)

    @functools.partial(
        pl.kernel,
        out_type=jax.ShapeDtypeStruct((hist, _D, nrows), jnp.float32),
        mesh=mesh,
        compiler_params=pltpu.CompilerParams(
            use_tc_tiling_on_sc=True, needs_layout_passes=False),
        scratch_types=[
            pltpu.VMEM((2, _TI), jnp.int32),
            pltpu.VMEM((2, _TI, 2 * _D), jnp.float32),
            pltpu.VMEM((2, _D, _TI), jnp.float32),
            pltpu.SemaphoreType.DMA,
            pltpu.SemaphoreType.DMA,
            pltpu.SemaphoreType.DMA,
            pltpu.SemaphoreType.DMA,
        ],
    )
    def k(tok_hbm, tab_hbm, out_hbm, idx_v, rows_v, trans_v, *sems):
        gsems = sems[:2]
        osems = sems[2:]
        wid = lax.axis_index("s") * _NC + lax.axis_index("c")
        i_base = wid * i_per_w

        def start_gather(buf, blk, gsem):
            j = lax.shift_right_logical(blk, 2)
            ib = lax.bitwise_and(blk, ib_per_w - 1)
            i0 = i_base + ib * _TI
            pltpu.sync_copy(tok_hbm.at[j, pl.ds(i0, _TI)], idx_v.at[buf])
            pltpu.async_copy(tab_hbm.at[idx_v.at[buf]], rows_v.at[buf], gsem)

        start_gather(0, 0, gsems[0])

        lanes = lax.iota(jnp.int32, _LANES)

        @pl.loop(0, nblocks, step=2)
        def _outer(G):
            for b in range(2):
                g = G + b
                nb = 1 - b

                @pl.when(g + 1 < nblocks)
                def _start_next():
                    @pl.when(g >= 1)
                    def _drain():
                        pltpu.make_async_copy(
                            trans_v.at[nb],
                            out_hbm.at[0, :, pl.ds(i_base, _TI)],
                            osems[nb]).wait()
                    start_gather(nb, g + 1, gsems[nb])

                pltpu.make_async_copy(
                    tab_hbm.at[idx_v.at[b]], rows_v.at[b], gsems[b]).wait()

                @plsc.parallel_loop(0, _TI, unroll=16)
                def _tok(r):
                    cid = jnp.full((_LANES,), 0, jnp.int32) + r
                    for l in range(_D // _LANES):
                        rid = l * _LANES + lanes
                        val = rows_v[b, r, pl.ds(l * _LANES, _LANES)] * _SCALE
                        plsc.store_scatter(trans_v.at[b], [rid, cid], val)

                j = lax.shift_right_logical(g, 2)
                ib = lax.bitwise_and(g, ib_per_w - 1)
                i0 = i_base + ib * _TI
                pltpu.async_copy(
                    trans_v.at[b], out_hbm.at[j, :, pl.ds(i0, _TI)],
                    osems[b])

        for b in range(2):
            pltpu.make_async_copy(
                trans_v.at[b], out_hbm.at[0, :, pl.ds(i_base, _TI)],
                osems[b]).wait()

    return k(tok_t, table_pad)


def kernel(tokens, table):
    tok_t = tokens.astype(jnp.int32).T
    table_pad = jnp.pad(table, ((0, 0), (0, _D)))
    out = _embed_lookup(tok_t, table_pad)
    return jnp.transpose(out, (2, 0, 1))


# speedup vs baseline: 2.9395x; 1.2562x over previous
import functools
import math

import jax
import jax.numpy as jnp
from jax import lax
from jax.experimental import pallas as pl
from jax.experimental.pallas import tpu as pltpu
from jax.experimental.pallas import tpu_sc as plsc

_D = 64
_NC, _NS = 2, 16
_NW = _NC * _NS
_LANES = 16
_SCALE = math.sqrt(_D)


@functools.partial(jax.jit, static_argnames=("B", "C"))
def _embed_lookup(tokens_flat, table, *, B, C):
    b_per_w = B // _NW
    nchunks = b_per_w // C
    assert nchunks % 2 == 0
    mesh = plsc.VectorSubcoreMesh(
        core_axis_name="c", subcore_axis_name="s",
        num_cores=_NC, num_subcores=_NS)

    @functools.partial(
        pl.kernel,
        out_type=jax.ShapeDtypeStruct((B, _D), jnp.float32),
        mesh=mesh,
        compiler_params=pltpu.CompilerParams(use_tc_tiling_on_sc=False),
        scratch_types=[
            pltpu.VMEM((2, C), jnp.int32),
            pltpu.VMEM((2, C, _D), jnp.float32),
            pltpu.SemaphoreType.DMA,
            pltpu.SemaphoreType.DMA,
            pltpu.SemaphoreType.DMA,
            pltpu.SemaphoreType.DMA,
        ],
    )
    def k(tokens_hbm, table_hbm, out_hbm, idx_v, rows_v, gs0, gs1, os0, os1):
        gsems = (gs0, gs1)
        osems = (os0, os1)
        wid = lax.axis_index("s") * _NC + lax.axis_index("c")
        base = wid * b_per_w

        def start_gather(buf, g, gsem):
            off = base + g * C
            pltpu.sync_copy(tokens_hbm.at[pl.ds(off, C)], idx_v.at[buf])
            pltpu.async_copy(table_hbm.at[idx_v.at[buf]], rows_v.at[buf], gsem)

        start_gather(0, 0, gs0)

        @pl.loop(0, nchunks, step=2)
        def _outer(G):
            for b in range(2):
                g = G + b
                nb = 1 - b

                @pl.when(g + 1 < nchunks)
                def _start_next():
                    @pl.when(g >= 1)
                    def _drain():
                        pltpu.make_async_copy(
                            rows_v.at[nb], out_hbm.at[pl.ds(base, C)],
                            osems[nb]).wait()
                    start_gather(nb, g + 1, gsems[nb])

                pltpu.make_async_copy(
                    table_hbm.at[idx_v.at[b]], rows_v.at[b], gsems[b]).wait()

                @pl.loop(0, C, unroll=8)
                def _scale(r):
                    for j in range(_D // _LANES):
                        sl = pl.ds(j * _LANES, _LANES)
                        rows_v[b, r, sl] = rows_v[b, r, sl] * _SCALE

                pltpu.async_copy(
                    rows_v.at[b], out_hbm.at[pl.ds(base + g * C, C)], osems[b])

        pltpu.make_async_copy(
            rows_v.at[0], out_hbm.at[pl.ds(base, C)], os0).wait()
        pltpu.make_async_copy(
            rows_v.at[1], out_hbm.at[pl.ds(base, C)], os1).wait()

    return k(tokens_flat, table)


def kernel(tokens, table):
    B = tokens.shape[0] * tokens.shape[1]
    flat = tokens.reshape(B).astype(jnp.int32)
    out = _embed_lookup(flat, table, B=B, C=512)
    return out.reshape(tokens.shape[0], tokens.shape[1], _D)
